# baseline (speedup 1.0000x reference)
import jax
import jax.numpy as jnp
from jax import lax
from jax.experimental import pallas as pl
from jax.experimental.pallas import tpu as pltpu
from jax.experimental.pallas import tpu_sc as plsc

NC = 2
NS = 16
NW = NC * NS
LANES = 16

D = 64
R = 128
NBUF = 4
GDEPTH = 2


def _rsqrt(x):
    i = lax.bitcast_convert_type(x, jnp.int32)
    i = jnp.int32(0x5F3759DF) - lax.shift_right_arithmetic(i, 1)
    y = lax.bitcast_convert_type(i, jnp.float32)
    for _ in range(3):
        y = y * (1.5 - 0.5 * x * y * y)
    return y


def _layernorm_block(buf, mi_s, is_s, gk, bk):
    iota = lax.iota(jnp.int32, LANES)

    def group(g, carry):
        row0 = g * LANES
        rows_idx = row0 + iota
        zeros = jnp.zeros((LANES,), jnp.float32)
        s = [zeros] * 4
        q = [zeros] * 4
        for d in range(D):
            c = plsc.load_gather(buf, [rows_idx, jnp.full((LANES,), d, jnp.int32)])
            p = d % 4
            s[p] = s[p] + c
            q[p] = q[p] + c * c
        ssum = (s[0] + s[1]) + (s[2] + s[3])
        qsum = (q[0] + q[1]) + (q[2] + q[3])
        mean = ssum * (1.0 / D)
        var = qsum * (1.0 / D) - mean * mean
        istd = _rsqrt(var + 1e-5)
        is_s[pl.ds(LANES, LANES)] = istd
        mi_s[pl.ds(LANES, LANES)] = mean * istd
        for r in range(LANES):
            lane = jnp.full((LANES,), LANES + r, jnp.int32)
            iv = plsc.load_gather(is_s, [lane])
            mv = plsc.load_gather(mi_s, [lane])
            for k in range(4):
                v = buf[row0 + r, pl.ds(k * LANES, LANES)]
                y = v * iv - mv
                buf[row0 + r, pl.ds(k * LANES, LANES)] = y * gk[k] + bk[k]
        return carry

    lax.fori_loop(0, R // LANES, group, None)


def _body(idx_hbm, table, gam, bet, out_hbm,
          idx_v, b0, b1, b2, b3, gam_v, bet_v, mi_s, is_s,
          g0, g1, g2, g3, o0, o1, o2, o3):
    nblk = idx_v.shape[0]
    cid = lax.axis_index("c")
    sid = lax.axis_index("s")
    wid = sid * NC + cid
    blk0 = wid * nblk

    pltpu.sync_copy(idx_hbm.at[pl.ds(blk0, nblk)], idx_v)
    pltpu.sync_copy(gam, gam_v)
    pltpu.sync_copy(bet, bet_v)

    bufs = [b0, b1, b2, b3]
    gsems = [g0, g1, g2, g3]
    osems = [o0, o1, o2, o3]

    gk = [gam_v[pl.ds(k * LANES, LANES)] for k in range(4)]
    bk = [bet_v[pl.ds(k * LANES, LANES)] for k in range(4)]

    for j in range(GDEPTH):
        pltpu.async_copy(table.at[idx_v.at[j]], bufs[j], gsems[j])

    def out_slice(j):
        return out_hbm.at[pl.ds((blk0 + j) * R, R)]

    def outer(o, carry):
        for b in range(NBUF):
            j = o * NBUF + b
            pltpu.make_async_copy(table.at[idx_v.at[j]], bufs[b], gsems[b]).wait()
            _layernorm_block(bufs[b], mi_s, is_s, gk, bk)
            pltpu.async_copy(bufs[b], out_slice(j), osems[b])
            nb = (b + GDEPTH) % NBUF
            nj = j + GDEPTH
            if b < GDEPTH:
                @pl.when(o > 0)
                def _wait():
                    pltpu.make_async_copy(bufs[nb], out_slice(j), osems[nb]).wait()

                pltpu.async_copy(table.at[idx_v.at[nj]], bufs[nb], gsems[nb])
            else:
                @pl.when(o < nblk // NBUF - 1)
                def _next():
                    pltpu.make_async_copy(bufs[nb], out_slice(j), osems[nb]).wait()
                    pltpu.async_copy(table.at[idx_v.at[nj]], bufs[nb], gsems[nb])
        return carry

    lax.fori_loop(0, nblk // NBUF, outer, None)

    for b in range(NBUF):
        pltpu.make_async_copy(bufs[b], out_slice(b), osems[b]).wait()


def kernel(x, table, ln_gamma, ln_beta):
    Bb, L = x.shape
    n = Bb * L
    d = table.shape[1]
    nblk = n // NW // R
    idx = x.reshape(n // R, R).astype(jnp.int32)

    mesh = plsc.VectorSubcoreMesh(core_axis_name="c", subcore_axis_name="s",
                                  num_cores=NC, num_subcores=NS)
    fn = pl.kernel(
        _body,
        out_type=jax.ShapeDtypeStruct((n, d), jnp.float32),
        mesh=mesh,
        compiler_params=pltpu.CompilerParams(
            needs_layout_passes=False, use_tc_tiling_on_sc=False),
        scratch_types=[
            pltpu.VMEM((nblk, R), jnp.int32),
            pltpu.VMEM((R, d), jnp.float32),
            pltpu.VMEM((R, d), jnp.float32),
            pltpu.VMEM((R, d), jnp.float32),
            pltpu.VMEM((R, d), jnp.float32),
            pltpu.VMEM((d,), jnp.float32),
            pltpu.VMEM((d,), jnp.float32),
            pltpu.VMEM((2 * LANES,), jnp.float32),
            pltpu.VMEM((2 * LANES,), jnp.float32),
        ] + [pltpu.SemaphoreType.DMA] * 8,
    )
    out = fn(idx, table, ln_gamma, ln_beta)
    return out.reshape(Bb, L, d)
